# Initial kernel scaffold; baseline (speedup 1.0000x reference)
import jax
import jax.numpy as jnp
from jax.experimental import pallas as pl


def kernel(kernel_x, design_x, enc_W, enc_b, conv_W0, conv_b0, ln_w0, ln_b0, conv_W1, conv_b1, ln_w1, ln_b1, conv_W2, conv_b2, ln_w2, ln_b2, head_W1, head_b1, head_W2, head_b2, head_W3, head_b3, kernel_edge_index, design_edge_index, pragma_count):
    raise NotImplementedError("write your pallas kernel here")


# speedup vs baseline: 12.7542x; 12.7542x over previous
import functools

import jax
import jax.numpy as jnp
from jax import lax
from jax.experimental import pallas as pl
from jax.experimental.pallas import tpu as pltpu
from jax.experimental.pallas import tpu_sc as plsc

_N = 10000
_E = 320000
_D = 128
_H = 128
_NC = 2
_NS = 16
_NW = _NC * _NS
_EPT = _E // _NW
_CH = 100
_NSTEP = _EPT // _CH
_CK = 400
_NCHUNK = _N // _CK
_ZR = 80

_R = 2000
_NB = 2 * _N // _R
_NBG = _NB // 2

_f32 = jnp.float32
_mesh = plsc.VectorSubcoreMesh(core_axis_name="c", subcore_axis_name="s")


@functools.partial(
    pl.kernel,
    out_type=jax.ShapeDtypeStruct((_NC, 2 * _N, _H), jnp.float32),
    mesh=_mesh,
    scratch_types=[
        pltpu.VMEM((_NSTEP, _CH), jnp.int32),
        pltpu.VMEM((_NSTEP, _CH), jnp.int32),
        pltpu.VMEM((_CH, _H), jnp.float32),
        pltpu.VMEM((_ZR, _H), jnp.float32),
        pltpu.VMEM_SHARED((_N, _H), jnp.float32),
        pltpu.SemaphoreType.DMA,
    ],
)
def _sc_scatter(g_hbm, src_hbm, dst_hbm, out_hbm, srcv, dstv, buf, zv, acc, sem):
    c = lax.axis_index("c")
    s = lax.axis_index("s")
    wid = s * _NC + c

    def _zrow(r, carry):
        for cc in range(_H // 16):
            zv[r, pl.ds(cc * 16, 16)] = jnp.zeros((16,), jnp.float32)
        return carry
    lax.fori_loop(0, _ZR, _zrow, 0)

    for gi in range(2):
        for rep in range(2):
            j = s + rep * _NS

            @pl.when(j < _NCHUNK)
            def _zero(j=j):
                for t in range(_CK // _ZR):
                    pltpu.sync_copy(zv, acc.at[pl.ds(j * _CK + t * _ZR, _ZR)])
        plsc.subcore_barrier()

        pltpu.sync_copy(src_hbm.at[gi, wid], srcv)
        pltpu.sync_copy(dst_hbm.at[gi, wid], dstv)

        def _step(j, carry):
            pltpu.async_copy(g_hbm.at[srcv.at[j]], buf, sem).wait()
            pltpu.sync_copy(buf, acc.at[dstv.at[j]], add=True)
            return carry
        lax.fori_loop(0, _NSTEP, _step, 0)
        plsc.subcore_barrier()

        for rep in range(2):
            j = s + rep * _NS

            @pl.when(j < _NCHUNK)
            def _wb(j=j, gi=gi):
                pltpu.sync_copy(acc.at[pl.ds(j * _CK, _CK)],
                                out_hbm.at[c, pl.ds(gi * _N + j * _CK, _CK)])


@functools.partial(
    pl.kernel,
    out_type=jax.ShapeDtypeStruct((_NC, 2 * _N, _H), jnp.float32),
    mesh=_mesh,
    scratch_types=[
        pltpu.VMEM((_NSTEP, _CH), jnp.int32),
        pltpu.VMEM((_CH, _H), jnp.float32),
        pltpu.VMEM((_ZR, _H), jnp.float32),
        pltpu.VMEM_SHARED((_N, _H), jnp.float32),
    ],
)
def _sc_degree(dst_hbm, out_hbm, dstv, ones, zv, acc):
    c = lax.axis_index("c")
    s = lax.axis_index("s")
    wid = s * _NC + c

    def _fill(r, carry):
        for cc in range(_H // 16):
            ones[r, pl.ds(cc * 16, 16)] = jnp.ones((16,), jnp.float32)
        return carry
    lax.fori_loop(0, _CH, _fill, 0)

    def _zrow(r, carry):
        for cc in range(_H // 16):
            zv[r, pl.ds(cc * 16, 16)] = jnp.zeros((16,), jnp.float32)
        return carry
    lax.fori_loop(0, _ZR, _zrow, 0)

    for gi in range(2):
        for rep in range(2):
            j = s + rep * _NS

            @pl.when(j < _NCHUNK)
            def _zero(j=j):
                for t in range(_CK // _ZR):
                    pltpu.sync_copy(zv, acc.at[pl.ds(j * _CK + t * _ZR, _ZR)])
        plsc.subcore_barrier()

        pltpu.sync_copy(dst_hbm.at[gi, wid], dstv)

        def _step(j, carry):
            pltpu.sync_copy(ones, acc.at[dstv.at[j]], add=True)
            return carry
        lax.fori_loop(0, _NSTEP, _step, 0)
        plsc.subcore_barrier()

        for rep in range(2):
            j = s + rep * _NS

            @pl.when(j < _NCHUNK)
            def _wb(j=j, gi=gi):
                pltpu.sync_copy(acc.at[pl.ds(j * _CK, _CK)],
                                out_hbm.at[c, pl.ds(gi * _N + j * _CK, _CK)])


def _tc_prep_body(degp_ref, x2_ref, encW_ref, encb_ref, W0_ref,
                  dinv_ref, mm_ref, g_ref):
    deg = degp_ref[0] + degp_ref[1] + 1.0
    dinv = 1.0 / jnp.sqrt(deg)
    dinv_ref[...] = dinv
    h0 = jnp.dot(x2_ref[...], encW_ref[...],
                 preferred_element_type=jnp.float32) + encb_ref[...]
    mm0 = jnp.dot(h0, W0_ref[...], preferred_element_type=jnp.float32)
    mm_ref[...] = mm0
    g_ref[...] = mm0 * dinv


def _tc_conv_body(s_ref, mm_ref, dinv_ref, b_ref, conv_ref, st1_ref, st2_ref):
    dinv = dinv_ref[...]
    conv = ((s_ref[0] + s_ref[1]) * dinv
            + mm_ref[...] * (dinv * dinv) + b_ref[...])
    conv_ref[...] = conv
    lane = lax.broadcasted_iota(jnp.int32, (1, _H), 1)
    psum = jnp.sum(conv)
    psq = jnp.sum(conv * conv)
    st1_ref[0] = jnp.where(lane == 0, psum,
                           jnp.where(lane == 1, psq, 0.0))
    st2_ref[0] = jnp.sum(conv, axis=0, keepdims=True)


def _ln_scale(st1, lo):
    tot = jnp.sum(st1[lo:lo + _NBG, :, 0:1])
    totsq = jnp.sum(st1[lo:lo + _NBG, :, 1:2])
    m = tot * (1.0 / (_N * _H))
    v = totsq * (1.0 / (_N * _H)) - m * m
    return m, 1.0 / jnp.sqrt(v + 1e-5)


def _tc_ln_mm_body(conv_ref, st1_ref, dinv_ref, lnw_ref, lnb_ref, Wn_ref,
                   mmn_ref, g_ref):
    i = pl.program_id(0)
    st1 = st1_ref[...]
    m_k, r_k = _ln_scale(st1, 0)
    m_d, r_d = _ln_scale(st1, _NBG)
    is_k = i < _NBG
    m = jnp.where(is_k, m_k, m_d)
    r = jnp.where(is_k, r_k, r_d)
    y = (conv_ref[...] - m) * r * lnw_ref[...] + lnb_ref[...]
    h = jnp.maximum(y, 0.0)
    mmn = jnp.dot(h, Wn_ref[...], preferred_element_type=jnp.float32)
    mmn_ref[...] = mmn
    g_ref[...] = mmn * dinv_ref[...]


def _tc_head_body(st1_ref, st2_ref, lnw_ref, lnb_ref,
                  hW1_ref, hb1_ref, hW2_ref, hb2_ref, hW3_ref, hb3_ref,
                  out_ref):
    st1 = st1_ref[...]
    pooled = []
    for lo in (0, _NBG):
        m, r = _ln_scale(st1, lo)
        colsum = jnp.sum(st2_ref[lo:lo + _NBG, 0], axis=0, keepdims=True)
        pooled.append((colsum - _N * m) * r * lnw_ref[...]
                      + _N * lnb_ref[...])
    h1 = jnp.maximum(
        jnp.dot(pooled[0], hW1_ref[0:_H], preferred_element_type=jnp.float32)
        + jnp.dot(pooled[1], hW1_ref[_H:2 * _H],
                  preferred_element_type=jnp.float32)
        + hb1_ref[...], 0.0)
    h2 = jnp.maximum(
        jnp.dot(h1, hW2_ref[...], preferred_element_type=jnp.float32)
        + hb2_ref[...], 0.0)
    out_ref[...] = (jnp.dot(h2, hW3_ref[...], preferred_element_type=jnp.float32)
                    + hb3_ref[...])


def _row_block(i):
    return (i, 0)


_vec_spec = pl.BlockSpec((_R, 1), _row_block)
_mat_spec = pl.BlockSpec((_R, _H), _row_block)
_full = pl.BlockSpec(index_map=lambda i: (0, 0))
_full1 = pl.BlockSpec(index_map=lambda i: (0,))


def _tc_prep(degp, x2, enc_W, enc_b, W0):
    return pl.pallas_call(
        _tc_prep_body,
        grid=(_NB,),
        in_specs=[pl.BlockSpec((_NC, _R, 1), lambda i: (0, i, 0)),
                  pl.BlockSpec((_R, _D), _row_block), _full, _full1, _full],
        out_specs=[_vec_spec, _mat_spec, _mat_spec],
        out_shape=[
            jax.ShapeDtypeStruct((2 * _N, 1), _f32),
            jax.ShapeDtypeStruct((2 * _N, _H), _f32),
            jax.ShapeDtypeStruct((2 * _N, _H), _f32),
        ],
    )(degp, x2, enc_W, enc_b, W0)


def _tc_conv(s_all, mm, dinv, b):
    return pl.pallas_call(
        _tc_conv_body,
        grid=(_NB,),
        in_specs=[pl.BlockSpec((_NC, _R, _H), lambda i: (0, i, 0)),
                  _mat_spec, _vec_spec, _full1],
        out_specs=[_mat_spec,
                   pl.BlockSpec((1, 1, _H), lambda i: (i, 0, 0)),
                   pl.BlockSpec((1, 1, _H), lambda i: (i, 0, 0))],
        out_shape=[
            jax.ShapeDtypeStruct((2 * _N, _H), _f32),
            jax.ShapeDtypeStruct((_NB, 1, _H), _f32),
            jax.ShapeDtypeStruct((_NB, 1, _H), _f32),
        ],
    )(s_all, mm, dinv, b)


def _tc_ln_mm(conv, st1, dinv, lnw, lnb, Wn):
    return pl.pallas_call(
        _tc_ln_mm_body,
        grid=(_NB,),
        in_specs=[_mat_spec, pl.BlockSpec(index_map=lambda i: (0, 0, 0)),
                  _vec_spec, _full1, _full1, _full],
        out_specs=[_mat_spec, _mat_spec],
        out_shape=[
            jax.ShapeDtypeStruct((2 * _N, _H), _f32),
            jax.ShapeDtypeStruct((2 * _N, _H), _f32),
        ],
    )(conv, st1, dinv, lnw, lnb, Wn)


def _tc_head(st1, st2, lnw, lnb, hW1, hb1, hW2, hb2, hW3, hb3):
    return pl.pallas_call(
        _tc_head_body,
        out_shape=jax.ShapeDtypeStruct((1, 1), _f32),
    )(st1, st2, lnw, lnb, hW1, hb1, hW2, hb2, hW3, hb3)


def kernel(kernel_x, design_x, enc_W, enc_b, conv_W0, conv_b0, ln_w0, ln_b0,
           conv_W1, conv_b1, ln_w1, ln_b1, conv_W2, conv_b2, ln_w2, ln_b2,
           head_W1, head_b1, head_W2, head_b2, head_W3, head_b3,
           kernel_edge_index, design_edge_index, pragma_count):
    srck = kernel_edge_index[0].reshape(_NW, _NSTEP, _CH)
    dstk = kernel_edge_index[1].reshape(_NW, _NSTEP, _CH)
    srcd = design_edge_index[0].reshape(_NW, _NSTEP, _CH)
    dstd = design_edge_index[1].reshape(_NW, _NSTEP, _CH)

    src2 = jnp.stack([srck, srcd + _N])
    dst2 = jnp.stack([dstk, dstd])

    degp = _sc_degree(dst2)[:, :, 0:1]

    x2 = jnp.concatenate([kernel_x, design_x], axis=0)
    dinv, mm, g = _tc_prep(degp, x2, enc_W, enc_b, conv_W0)

    Ws = (conv_W1, conv_W2)
    bs = (conv_b0, conv_b1, conv_b2)
    lnws = (ln_w0, ln_w1, ln_w2)
    lnbs = (ln_b0, ln_b1, ln_b2)
    for i in range(3):
        s_all = _sc_scatter(g, src2, dst2)
        conv, st1, st2 = _tc_conv(s_all, mm, dinv, bs[i])
        if i < 2:
            mm, g = _tc_ln_mm(conv, st1, dinv, lnws[i], lnbs[i], Ws[i])
        else:
            out = _tc_head(st1, st2, lnws[2], lnbs[2],
                           head_W1, head_b1, head_W2, head_b2,
                           head_W3, head_b3)
    return out


# speedup vs baseline: 16.0298x; 1.2568x over previous
import functools

import jax
import jax.numpy as jnp
from jax import lax
from jax.experimental import pallas as pl
from jax.experimental.pallas import tpu as pltpu
from jax.experimental.pallas import tpu_sc as plsc

_N = 10000
_E = 320000
_D = 128
_H = 128
_NC = 2
_NS = 16
_NW = _NC * _NS
_EPT = _E // _NW
_CH = 100
_NSTEP = _EPT // _CH
_CK = 400
_NCHUNK = _N // _CK
_ZR = 80

_R = 2000
_NB = 2 * _N // _R
_NBG = _NB // 2

_f32 = jnp.float32
_mesh = plsc.VectorSubcoreMesh(core_axis_name="c", subcore_axis_name="s")


@functools.partial(
    pl.kernel,
    out_type=jax.ShapeDtypeStruct((_NC, 2 * _N, _H), jnp.float32),
    mesh=_mesh,
    scratch_types=[
        pltpu.VMEM((_NSTEP // 2, _CH), jnp.int32),
        pltpu.VMEM((_NSTEP // 2, _CH), jnp.int32),
        pltpu.VMEM((_CH, _H), jnp.float32),
        pltpu.VMEM((_CH, _H), jnp.float32),
        pltpu.VMEM_SHARED((_N, _H), jnp.float32),
        pltpu.SemaphoreType.DMA,
    ],
)
def _sc_scatter(g_hbm, src_hbm, dst_hbm, out_hbm, srcv, dstv, bufa, bufb,
                acc, sem):
    c = lax.axis_index("c")
    s = lax.axis_index("s")
    wid = s * _NC + c

    for gi in range(2):
        def _zrow(r, carry):
            for cc in range(_H // 16):
                bufa[r, pl.ds(cc * 16, 16)] = jnp.zeros((16,), jnp.float32)
            return carry
        lax.fori_loop(0, _CH, _zrow, 0)
        for rep in range(2):
            j = s + rep * _NS

            @pl.when(j < _NCHUNK)
            def _zero(j=j):
                for t in range(_CK // _CH):
                    pltpu.sync_copy(bufa, acc.at[pl.ds(j * _CK + t * _CH, _CH)])
        plsc.subcore_barrier()

        for hf in range(2):
            pltpu.sync_copy(src_hbm.at[gi, wid, hf], srcv)
            pltpu.sync_copy(dst_hbm.at[gi, wid, hf], dstv)
            pltpu.async_copy(g_hbm.at[srcv.at[0]], bufa, sem)

            def _pair(p, carry):
                j = 2 * p
                pltpu.make_async_copy(g_hbm.at[srcv.at[j]], bufa, sem).wait()
                pltpu.async_copy(g_hbm.at[srcv.at[j + 1]], bufb, sem)
                pltpu.sync_copy(bufa, acc.at[dstv.at[j]], add=True)
                pltpu.make_async_copy(g_hbm.at[srcv.at[j + 1]], bufb, sem).wait()

                @pl.when(p + 1 < _NSTEP // 4)
                def _next():
                    pltpu.async_copy(g_hbm.at[srcv.at[j + 2]], bufa, sem)
                pltpu.sync_copy(bufb, acc.at[dstv.at[j + 1]], add=True)
                return carry
            lax.fori_loop(0, _NSTEP // 4, _pair, 0)
        plsc.subcore_barrier()

        for rep in range(2):
            j = s + rep * _NS

            @pl.when(j < _NCHUNK)
            def _wb(j=j, gi=gi):
                pltpu.sync_copy(acc.at[pl.ds(j * _CK, _CK)],
                                out_hbm.at[c, pl.ds(gi * _N + j * _CK, _CK)])


@functools.partial(
    pl.kernel,
    out_type=jax.ShapeDtypeStruct((_NC, 2 * _N, _H), jnp.float32),
    mesh=_mesh,
    scratch_types=[
        pltpu.VMEM((_NSTEP, _CH), jnp.int32),
        pltpu.VMEM((_CH, _H), jnp.float32),
        pltpu.VMEM((_ZR, _H), jnp.float32),
        pltpu.VMEM_SHARED((_N, _H), jnp.float32),
    ],
)
def _sc_degree(dst_hbm, out_hbm, dstv, ones, zv, acc):
    c = lax.axis_index("c")
    s = lax.axis_index("s")
    wid = s * _NC + c

    def _fill(r, carry):
        for cc in range(_H // 16):
            ones[r, pl.ds(cc * 16, 16)] = jnp.ones((16,), jnp.float32)
        return carry
    lax.fori_loop(0, _CH, _fill, 0)

    def _zrow(r, carry):
        for cc in range(_H // 16):
            zv[r, pl.ds(cc * 16, 16)] = jnp.zeros((16,), jnp.float32)
        return carry
    lax.fori_loop(0, _ZR, _zrow, 0)

    for gi in range(2):
        for rep in range(2):
            j = s + rep * _NS

            @pl.when(j < _NCHUNK)
            def _zero(j=j):
                for t in range(_CK // _ZR):
                    pltpu.sync_copy(zv, acc.at[pl.ds(j * _CK + t * _ZR, _ZR)])
        plsc.subcore_barrier()

        pltpu.sync_copy(dst_hbm.at[gi, wid], dstv)

        def _step(j, carry):
            pltpu.sync_copy(ones, acc.at[dstv.at[j]], add=True)
            return carry
        lax.fori_loop(0, _NSTEP, _step, 0)
        plsc.subcore_barrier()

        for rep in range(2):
            j = s + rep * _NS

            @pl.when(j < _NCHUNK)
            def _wb(j=j, gi=gi):
                pltpu.sync_copy(acc.at[pl.ds(j * _CK, _CK)],
                                out_hbm.at[c, pl.ds(gi * _N + j * _CK, _CK)])


def _tc_prep_body(degp_ref, x2_ref, encW_ref, encb_ref, W0_ref,
                  dinv_ref, mm_ref, g_ref):
    deg = degp_ref[0] + degp_ref[1] + 1.0
    dinv = 1.0 / jnp.sqrt(deg)
    dinv_ref[...] = dinv
    h0 = jnp.dot(x2_ref[...], encW_ref[...],
                 preferred_element_type=jnp.float32) + encb_ref[...]
    mm0 = jnp.dot(h0, W0_ref[...], preferred_element_type=jnp.float32)
    mm_ref[...] = mm0
    g_ref[...] = mm0 * dinv


def _tc_conv_body(s_ref, mm_ref, dinv_ref, b_ref, conv_ref, st1_ref, st2_ref):
    dinv = dinv_ref[...]
    conv = ((s_ref[0] + s_ref[1]) * dinv
            + mm_ref[...] * (dinv * dinv) + b_ref[...])
    conv_ref[...] = conv
    lane = lax.broadcasted_iota(jnp.int32, (1, _H), 1)
    psum = jnp.sum(conv)
    psq = jnp.sum(conv * conv)
    st1_ref[0] = jnp.where(lane == 0, psum,
                           jnp.where(lane == 1, psq, 0.0))
    st2_ref[0] = jnp.sum(conv, axis=0, keepdims=True)


def _ln_scale(st1, lo):
    tot = jnp.sum(st1[lo:lo + _NBG, :, 0:1])
    totsq = jnp.sum(st1[lo:lo + _NBG, :, 1:2])
    m = tot * (1.0 / (_N * _H))
    v = totsq * (1.0 / (_N * _H)) - m * m
    return m, 1.0 / jnp.sqrt(v + 1e-5)


def _tc_ln_mm_body(conv_ref, st1_ref, dinv_ref, lnw_ref, lnb_ref, Wn_ref,
                   mmn_ref, g_ref):
    i = pl.program_id(0)
    st1 = st1_ref[...]
    m_k, r_k = _ln_scale(st1, 0)
    m_d, r_d = _ln_scale(st1, _NBG)
    is_k = i < _NBG
    m = jnp.where(is_k, m_k, m_d)
    r = jnp.where(is_k, r_k, r_d)
    y = (conv_ref[...] - m) * r * lnw_ref[...] + lnb_ref[...]
    h = jnp.maximum(y, 0.0)
    mmn = jnp.dot(h, Wn_ref[...], preferred_element_type=jnp.float32)
    mmn_ref[...] = mmn
    g_ref[...] = mmn * dinv_ref[...]


def _tc_head_body(st1_ref, st2_ref, lnw_ref, lnb_ref,
                  hW1_ref, hb1_ref, hW2_ref, hb2_ref, hW3_ref, hb3_ref,
                  out_ref):
    st1 = st1_ref[...]
    pooled = []
    for lo in (0, _NBG):
        m, r = _ln_scale(st1, lo)
        colsum = jnp.sum(st2_ref[lo:lo + _NBG, 0], axis=0, keepdims=True)
        pooled.append((colsum - _N * m) * r * lnw_ref[...]
                      + _N * lnb_ref[...])
    h1 = jnp.maximum(
        jnp.dot(pooled[0], hW1_ref[0:_H], preferred_element_type=jnp.float32)
        + jnp.dot(pooled[1], hW1_ref[_H:2 * _H],
                  preferred_element_type=jnp.float32)
        + hb1_ref[...], 0.0)
    h2 = jnp.maximum(
        jnp.dot(h1, hW2_ref[...], preferred_element_type=jnp.float32)
        + hb2_ref[...], 0.0)
    out_ref[...] = (jnp.dot(h2, hW3_ref[...], preferred_element_type=jnp.float32)
                    + hb3_ref[...])


def _row_block(i):
    return (i, 0)


_vec_spec = pl.BlockSpec((_R, 1), _row_block)
_mat_spec = pl.BlockSpec((_R, _H), _row_block)
_full = pl.BlockSpec(index_map=lambda i: (0, 0))
_full1 = pl.BlockSpec(index_map=lambda i: (0,))


def _tc_prep(degp, x2, enc_W, enc_b, W0):
    return pl.pallas_call(
        _tc_prep_body,
        grid=(_NB,),
        in_specs=[pl.BlockSpec((_NC, _R, 1), lambda i: (0, i, 0)),
                  pl.BlockSpec((_R, _D), _row_block), _full, _full1, _full],
        out_specs=[_vec_spec, _mat_spec, _mat_spec],
        out_shape=[
            jax.ShapeDtypeStruct((2 * _N, 1), _f32),
            jax.ShapeDtypeStruct((2 * _N, _H), _f32),
            jax.ShapeDtypeStruct((2 * _N, _H), _f32),
        ],
    )(degp, x2, enc_W, enc_b, W0)


def _tc_conv(s_all, mm, dinv, b):
    return pl.pallas_call(
        _tc_conv_body,
        grid=(_NB,),
        in_specs=[pl.BlockSpec((_NC, _R, _H), lambda i: (0, i, 0)),
                  _mat_spec, _vec_spec, _full1],
        out_specs=[_mat_spec,
                   pl.BlockSpec((1, 1, _H), lambda i: (i, 0, 0)),
                   pl.BlockSpec((1, 1, _H), lambda i: (i, 0, 0))],
        out_shape=[
            jax.ShapeDtypeStruct((2 * _N, _H), _f32),
            jax.ShapeDtypeStruct((_NB, 1, _H), _f32),
            jax.ShapeDtypeStruct((_NB, 1, _H), _f32),
        ],
    )(s_all, mm, dinv, b)


def _tc_ln_mm(conv, st1, dinv, lnw, lnb, Wn):
    return pl.pallas_call(
        _tc_ln_mm_body,
        grid=(_NB,),
        in_specs=[_mat_spec, pl.BlockSpec(index_map=lambda i: (0, 0, 0)),
                  _vec_spec, _full1, _full1, _full],
        out_specs=[_mat_spec, _mat_spec],
        out_shape=[
            jax.ShapeDtypeStruct((2 * _N, _H), _f32),
            jax.ShapeDtypeStruct((2 * _N, _H), _f32),
        ],
    )(conv, st1, dinv, lnw, lnb, Wn)


def _tc_head(st1, st2, lnw, lnb, hW1, hb1, hW2, hb2, hW3, hb3):
    return pl.pallas_call(
        _tc_head_body,
        out_shape=jax.ShapeDtypeStruct((1, 1), _f32),
    )(st1, st2, lnw, lnb, hW1, hb1, hW2, hb2, hW3, hb3)


def kernel(kernel_x, design_x, enc_W, enc_b, conv_W0, conv_b0, ln_w0, ln_b0,
           conv_W1, conv_b1, ln_w1, ln_b1, conv_W2, conv_b2, ln_w2, ln_b2,
           head_W1, head_b1, head_W2, head_b2, head_W3, head_b3,
           kernel_edge_index, design_edge_index, pragma_count):
    srck = kernel_edge_index[0].reshape(_NW, _NSTEP, _CH)
    dstk = kernel_edge_index[1].reshape(_NW, _NSTEP, _CH)
    srcd = design_edge_index[0].reshape(_NW, _NSTEP, _CH)
    dstd = design_edge_index[1].reshape(_NW, _NSTEP, _CH)

    src2 = jnp.stack([srck, srcd + _N])
    dst2 = jnp.stack([dstk, dstd])
    src2h = src2.reshape(2, _NW, 2, _NSTEP // 2, _CH)
    dst2h = dst2.reshape(2, _NW, 2, _NSTEP // 2, _CH)

    degp = _sc_degree(dst2)[:, :, 0:1]

    x2 = jnp.concatenate([kernel_x, design_x], axis=0)
    dinv, mm, g = _tc_prep(degp, x2, enc_W, enc_b, conv_W0)

    Ws = (conv_W1, conv_W2)
    bs = (conv_b0, conv_b1, conv_b2)
    lnws = (ln_w0, ln_w1, ln_w2)
    lnbs = (ln_b0, ln_b1, ln_b2)
    for i in range(3):
        s_all = _sc_scatter(g, src2h, dst2h)
        conv, st1, st2 = _tc_conv(s_all, mm, dinv, bs[i])
        if i < 2:
            mm, g = _tc_ln_mm(conv, st1, dinv, lnws[i], lnbs[i], Ws[i])
        else:
            out = _tc_head(st1, st2, lnws[2], lnbs[2],
                           head_W1, head_b1, head_W2, head_b2,
                           head_W3, head_b3)
    return out


# speedup vs baseline: 17.5706x; 1.0961x over previous
import functools

import jax
import jax.numpy as jnp
from jax import lax
from jax.experimental import pallas as pl
from jax.experimental.pallas import tpu as pltpu
from jax.experimental.pallas import tpu_sc as plsc

_N = 10000
_E = 320000
_D = 128
_H = 128
_NC = 2
_NS = 16
_NW = _NC * _NS
_EPT = _E // _NW
_CH = 100
_NSTEP = _EPT // _CH
_CK = 400
_NCHUNK = _N // _CK
_ZR = 80

_R = 2000
_NB = 2 * _N // _R
_NBG = _NB // 2

_f32 = jnp.float32
_mesh = plsc.VectorSubcoreMesh(core_axis_name="c", subcore_axis_name="s")


@functools.partial(
    pl.kernel,
    out_type=jax.ShapeDtypeStruct((_NC, 2 * _N, _H), jnp.float32),
    mesh=_mesh,
    scratch_types=[
        pltpu.VMEM((_NSTEP // 2, _CH), jnp.int32),
        pltpu.VMEM((_NSTEP // 2, _CH), jnp.int32),
        pltpu.VMEM((_CH, _H), jnp.float32),
        pltpu.VMEM((_CH, _H), jnp.float32),
        pltpu.VMEM_SHARED((_N, _H), jnp.float32),
        pltpu.SemaphoreType.DMA,
    ],
)
def _sc_scatter(g_hbm, src_hbm, dst_hbm, out_hbm, srcv, dstv, bufa, bufb,
                acc, sem):
    c = lax.axis_index("c")
    s = lax.axis_index("s")
    wid = s * _NC + c

    for gi in range(2):
        def _zrow(r, carry):
            for cc in range(_H // 16):
                bufa[r, pl.ds(cc * 16, 16)] = jnp.zeros((16,), jnp.float32)
            return carry
        lax.fori_loop(0, _CH, _zrow, 0)
        for rep in range(2):
            j = s + rep * _NS

            @pl.when(j < _NCHUNK)
            def _zero(j=j):
                for t in range(_CK // _CH):
                    pltpu.sync_copy(bufa, acc.at[pl.ds(j * _CK + t * _CH, _CH)])
        plsc.subcore_barrier()

        for hf in range(2):
            pltpu.sync_copy(src_hbm.at[gi, wid, hf], srcv)
            pltpu.sync_copy(dst_hbm.at[gi, wid, hf], dstv)
            pltpu.async_copy(g_hbm.at[srcv.at[0]], bufa, sem)

            def _pair(p, carry):
                j = 2 * p
                pltpu.make_async_copy(g_hbm.at[srcv.at[j]], bufa, sem).wait()
                pltpu.async_copy(g_hbm.at[srcv.at[j + 1]], bufb, sem)
                pltpu.sync_copy(bufa, acc.at[dstv.at[j]], add=True)
                pltpu.make_async_copy(g_hbm.at[srcv.at[j + 1]], bufb, sem).wait()

                @pl.when(p + 1 < _NSTEP // 4)
                def _next():
                    pltpu.async_copy(g_hbm.at[srcv.at[j + 2]], bufa, sem)
                pltpu.sync_copy(bufb, acc.at[dstv.at[j + 1]], add=True)
                return carry
            lax.fori_loop(0, _NSTEP // 4, _pair, 0)
        plsc.subcore_barrier()

        for rep in range(2):
            j = s + rep * _NS

            @pl.when(j < _NCHUNK)
            def _wb(j=j, gi=gi):
                pltpu.sync_copy(acc.at[pl.ds(j * _CK, _CK)],
                                out_hbm.at[c, pl.ds(gi * _N + j * _CK, _CK)])


_DEPT = 2 * _E // _NW
_DSEG = 800
_DNCH = 2 * _N // _DSEG


@functools.partial(
    pl.kernel,
    out_type=jax.ShapeDtypeStruct((_NC * 2 * _N,), jnp.float32),
    mesh=_mesh,
    compiler_params=pltpu.CompilerParams(needs_layout_passes=False),
    scratch_types=[
        pltpu.VMEM((_DEPT // 400, 400), jnp.int32),
        pltpu.VMEM((2 * _N,), jnp.float32),
        pltpu.VMEM((_NS * _DSEG,), jnp.float32),
        pltpu.VMEM((_DSEG,), jnp.float32),
        pltpu.VMEM_SHARED((_NS * 2 * _N,), jnp.float32),
    ],
)
def _sc_degree(dst_hbm, out_hbm, dstv, hist, red, outv, spbuf):
    c = lax.axis_index("c")
    s = lax.axis_index("s")
    wid = s * _NC + c
    zeros16 = jnp.zeros((16,), jnp.float32)
    ones16 = jnp.ones((16,), jnp.float32)

    def _zrow(r, carry):
        hist[pl.ds(r * 16, 16)] = zeros16
        return carry
    lax.fori_loop(0, 2 * _N // 16, _zrow, 0)

    pltpu.sync_copy(dst_hbm.at[wid], dstv)

    def _count(r, carry):
        for k in range(400 // 16):
            idx = dstv[r, pl.ds(k * 16, 16)]
            plsc.addupdate_scatter(hist, [idx], ones16)
        return carry
    lax.fori_loop(0, _DEPT // 400, _count, 0)

    pltpu.sync_copy(hist, spbuf.at[pl.ds(s * 2 * _N, 2 * _N)])
    plsc.subcore_barrier()

    for rep in range(2):
        j = s + rep * _NS

        @pl.when(j < _DNCH)
        def _reduce(j=j):
            for t in range(_NS):
                pltpu.sync_copy(
                    spbuf.at[pl.ds(t * 2 * _N + j * _DSEG, _DSEG)],
                    red.at[pl.ds(t * _DSEG, _DSEG)])

            def _sum(k, carry):
                v = red[pl.ds(k * 16, 16)]
                for t in range(1, _NS):
                    v = v + red[pl.ds(t * _DSEG + k * 16, 16)]
                outv[pl.ds(k * 16, 16)] = v
                return carry
            lax.fori_loop(0, _DSEG // 16, _sum, 0)
            pltpu.sync_copy(
                outv, out_hbm.at[pl.ds(c * 2 * _N + j * _DSEG, _DSEG)])


def _tc_prep_body(degp_ref, x2_ref, encW_ref, encb_ref, W0_ref,
                  dinv_ref, mm_ref, g_ref):
    deg = degp_ref[:, 0:1] + degp_ref[:, 1:2] + 1.0
    dinv = 1.0 / jnp.sqrt(deg)
    dinv_ref[...] = dinv
    h0 = jnp.dot(x2_ref[...], encW_ref[...],
                 preferred_element_type=jnp.float32) + encb_ref[...]
    mm0 = jnp.dot(h0, W0_ref[...], preferred_element_type=jnp.float32)
    mm_ref[...] = mm0
    g_ref[...] = mm0 * dinv


def _tc_conv_body(s_ref, mm_ref, dinv_ref, b_ref, conv_ref, st1_ref, st2_ref):
    dinv = dinv_ref[...]
    conv = ((s_ref[0] + s_ref[1]) * dinv
            + mm_ref[...] * (dinv * dinv) + b_ref[...])
    conv_ref[...] = conv
    lane = lax.broadcasted_iota(jnp.int32, (1, _H), 1)
    psum = jnp.sum(conv)
    psq = jnp.sum(conv * conv)
    st1_ref[0] = jnp.where(lane == 0, psum,
                           jnp.where(lane == 1, psq, 0.0))
    st2_ref[0] = jnp.sum(conv, axis=0, keepdims=True)


def _ln_scale(st1, lo):
    tot = jnp.sum(st1[lo:lo + _NBG, :, 0:1])
    totsq = jnp.sum(st1[lo:lo + _NBG, :, 1:2])
    m = tot * (1.0 / (_N * _H))
    v = totsq * (1.0 / (_N * _H)) - m * m
    return m, 1.0 / jnp.sqrt(v + 1e-5)


def _tc_ln_mm_body(conv_ref, st1_ref, dinv_ref, lnw_ref, lnb_ref, Wn_ref,
                   mmn_ref, g_ref):
    i = pl.program_id(0)
    st1 = st1_ref[...]
    m_k, r_k = _ln_scale(st1, 0)
    m_d, r_d = _ln_scale(st1, _NBG)
    is_k = i < _NBG
    m = jnp.where(is_k, m_k, m_d)
    r = jnp.where(is_k, r_k, r_d)
    y = (conv_ref[...] - m) * r * lnw_ref[...] + lnb_ref[...]
    h = jnp.maximum(y, 0.0)
    mmn = jnp.dot(h, Wn_ref[...], preferred_element_type=jnp.float32)
    mmn_ref[...] = mmn
    g_ref[...] = mmn * dinv_ref[...]


def _tc_head_body(st1_ref, st2_ref, lnw_ref, lnb_ref,
                  hW1_ref, hb1_ref, hW2_ref, hb2_ref, hW3_ref, hb3_ref,
                  out_ref):
    st1 = st1_ref[...]
    pooled = []
    for lo in (0, _NBG):
        m, r = _ln_scale(st1, lo)
        colsum = jnp.sum(st2_ref[lo:lo + _NBG, 0], axis=0, keepdims=True)
        pooled.append((colsum - _N * m) * r * lnw_ref[...]
                      + _N * lnb_ref[...])
    h1 = jnp.maximum(
        jnp.dot(pooled[0], hW1_ref[0:_H], preferred_element_type=jnp.float32)
        + jnp.dot(pooled[1], hW1_ref[_H:2 * _H],
                  preferred_element_type=jnp.float32)
        + hb1_ref[...], 0.0)
    h2 = jnp.maximum(
        jnp.dot(h1, hW2_ref[...], preferred_element_type=jnp.float32)
        + hb2_ref[...], 0.0)
    out_ref[...] = (jnp.dot(h2, hW3_ref[...], preferred_element_type=jnp.float32)
                    + hb3_ref[...])


def _row_block(i):
    return (i, 0)


_vec_spec = pl.BlockSpec((_R, 1), _row_block)
_mat_spec = pl.BlockSpec((_R, _H), _row_block)
_full = pl.BlockSpec(index_map=lambda i: (0, 0))
_full1 = pl.BlockSpec(index_map=lambda i: (0,))


def _tc_prep(degp, x2, enc_W, enc_b, W0):
    return pl.pallas_call(
        _tc_prep_body,
        grid=(_NB,),
        in_specs=[pl.BlockSpec((_R, _NC), _row_block),
                  pl.BlockSpec((_R, _D), _row_block), _full, _full1, _full],
        out_specs=[_vec_spec, _mat_spec, _mat_spec],
        out_shape=[
            jax.ShapeDtypeStruct((2 * _N, 1), _f32),
            jax.ShapeDtypeStruct((2 * _N, _H), _f32),
            jax.ShapeDtypeStruct((2 * _N, _H), _f32),
        ],
    )(degp, x2, enc_W, enc_b, W0)


def _tc_conv(s_all, mm, dinv, b):
    return pl.pallas_call(
        _tc_conv_body,
        grid=(_NB,),
        in_specs=[pl.BlockSpec((_NC, _R, _H), lambda i: (0, i, 0)),
                  _mat_spec, _vec_spec, _full1],
        out_specs=[_mat_spec,
                   pl.BlockSpec((1, 1, _H), lambda i: (i, 0, 0)),
                   pl.BlockSpec((1, 1, _H), lambda i: (i, 0, 0))],
        out_shape=[
            jax.ShapeDtypeStruct((2 * _N, _H), _f32),
            jax.ShapeDtypeStruct((_NB, 1, _H), _f32),
            jax.ShapeDtypeStruct((_NB, 1, _H), _f32),
        ],
    )(s_all, mm, dinv, b)


def _tc_ln_mm(conv, st1, dinv, lnw, lnb, Wn):
    return pl.pallas_call(
        _tc_ln_mm_body,
        grid=(_NB,),
        in_specs=[_mat_spec, pl.BlockSpec(index_map=lambda i: (0, 0, 0)),
                  _vec_spec, _full1, _full1, _full],
        out_specs=[_mat_spec, _mat_spec],
        out_shape=[
            jax.ShapeDtypeStruct((2 * _N, _H), _f32),
            jax.ShapeDtypeStruct((2 * _N, _H), _f32),
        ],
    )(conv, st1, dinv, lnw, lnb, Wn)


def _tc_head(st1, st2, lnw, lnb, hW1, hb1, hW2, hb2, hW3, hb3):
    return pl.pallas_call(
        _tc_head_body,
        out_shape=jax.ShapeDtypeStruct((1, 1), _f32),
    )(st1, st2, lnw, lnb, hW1, hb1, hW2, hb2, hW3, hb3)


def kernel(kernel_x, design_x, enc_W, enc_b, conv_W0, conv_b0, ln_w0, ln_b0,
           conv_W1, conv_b1, ln_w1, ln_b1, conv_W2, conv_b2, ln_w2, ln_b2,
           head_W1, head_b1, head_W2, head_b2, head_W3, head_b3,
           kernel_edge_index, design_edge_index, pragma_count):
    srck = kernel_edge_index[0].reshape(_NW, _NSTEP, _CH)
    dstk = kernel_edge_index[1].reshape(_NW, _NSTEP, _CH)
    srcd = design_edge_index[0].reshape(_NW, _NSTEP, _CH)
    dstd = design_edge_index[1].reshape(_NW, _NSTEP, _CH)

    src2 = jnp.stack([srck, srcd + _N])
    dst2 = jnp.stack([dstk, dstd])
    src2h = src2.reshape(2, _NW, 2, _NSTEP // 2, _CH)
    dst2h = dst2.reshape(2, _NW, 2, _NSTEP // 2, _CH)
    dstdeg = jnp.concatenate(
        [kernel_edge_index[1],
         design_edge_index[1] + _N]).reshape(_NW, _DEPT // 400, 400)

    degp = jnp.transpose(_sc_degree(dstdeg).reshape(_NC, 2 * _N))

    x2 = jnp.concatenate([kernel_x, design_x], axis=0)
    dinv, mm, g = _tc_prep(degp, x2, enc_W, enc_b, conv_W0)

    Ws = (conv_W1, conv_W2)
    bs = (conv_b0, conv_b1, conv_b2)
    lnws = (ln_w0, ln_w1, ln_w2)
    lnbs = (ln_b0, ln_b1, ln_b2)
    for i in range(3):
        s_all = _sc_scatter(g, src2h, dst2h)
        conv, st1, st2 = _tc_conv(s_all, mm, dinv, bs[i])
        if i < 2:
            mm, g = _tc_ln_mm(conv, st1, dinv, lnws[i], lnbs[i], Ws[i])
        else:
            out = _tc_head(st1, st2, lnws[2], lnbs[2],
                           head_W1, head_b1, head_W2, head_b2,
                           head_W3, head_b3)
    return out


# speedup vs baseline: 20.6515x; 1.1753x over previous
import functools

import jax
import jax.numpy as jnp
from jax import lax
from jax.experimental import pallas as pl
from jax.experimental.pallas import tpu as pltpu
from jax.experimental.pallas import tpu_sc as plsc

_N = 10000
_E = 320000
_D = 128
_H = 128
_NC = 2
_NS = 16
_NW = _NC * _NS
_EPT = _E // _NW
_CH = 100
_NSTEP = _EPT // _CH
_CK = 400
_NCHUNK = _N // _CK
_ZR = 80

_R = 2000
_NB = 2 * _N // _R
_NBG = _NB // 2

_f32 = jnp.float32
_mesh = plsc.VectorSubcoreMesh(core_axis_name="c", subcore_axis_name="s")


@functools.partial(
    pl.kernel,
    out_type=jax.ShapeDtypeStruct((_NC, 2 * _N, _H), jnp.float32),
    mesh=_mesh,
    scratch_types=[
        pltpu.VMEM((_NSTEP // 2, _CH), jnp.int32),
        pltpu.VMEM((_NSTEP // 2, _CH), jnp.int32),
        pltpu.VMEM((_CH, _H), jnp.float32),
        pltpu.VMEM((_CH, _H), jnp.float32),
        pltpu.VMEM_SHARED((_N, _H), jnp.float32),
        pltpu.SemaphoreType.DMA,
        pltpu.SemaphoreType.DMA,
    ],
)
def _sc_scatter(g_hbm, src_hbm, dst_hbm, out_hbm, srcv, dstv, bufa, bufb,
                acc, gsem, ssem):
    c = lax.axis_index("c")
    s = lax.axis_index("s")
    wid = s * _NC + c

    for gi in range(2):
        def _zrow(r, carry):
            for cc in range(_H // 16):
                bufa[r, pl.ds(cc * 16, 16)] = jnp.zeros((16,), jnp.float32)
            return carry
        lax.fori_loop(0, _CH, _zrow, 0)
        for rep in range(2):
            j = s + rep * _NS

            @pl.when(j < _NCHUNK)
            def _zero(j=j):
                for t in range(_CK // _CH):
                    pltpu.sync_copy(bufa, acc.at[pl.ds(j * _CK + t * _CH, _CH)])
        plsc.subcore_barrier()

        for hf in range(2):
            pltpu.sync_copy(src_hbm.at[gi, wid, hf], srcv)
            pltpu.sync_copy(dst_hbm.at[gi, wid, hf], dstv)
            half = _NSTEP // 2
            pltpu.async_copy(g_hbm.at[srcv.at[0]], bufa, gsem)
            pltpu.async_copy(g_hbm.at[srcv.at[1]], bufb, gsem)

            def _pair(p, carry):
                j = 2 * p
                pltpu.make_async_copy(g_hbm.at[srcv.at[j]], bufa, gsem).wait()
                pltpu.async_copy(bufa, acc.at[dstv.at[j]], ssem, add=True)
                pltpu.make_async_copy(g_hbm.at[srcv.at[j + 1]], bufb,
                                      gsem).wait()
                pltpu.async_copy(bufb, acc.at[dstv.at[j + 1]], ssem, add=True)
                pltpu.make_async_copy(bufa, acc.at[dstv.at[j]], ssem).wait()

                @pl.when(j + 2 < half)
                def _ga(j=j):
                    pltpu.async_copy(g_hbm.at[srcv.at[j + 2]], bufa, gsem)
                pltpu.make_async_copy(bufb, acc.at[dstv.at[j + 1]],
                                      ssem).wait()

                @pl.when(j + 3 < half)
                def _gb(j=j):
                    pltpu.async_copy(g_hbm.at[srcv.at[j + 3]], bufb, gsem)
                return carry
            lax.fori_loop(0, _NSTEP // 4, _pair, 0)
        plsc.subcore_barrier()

        for rep in range(2):
            j = s + rep * _NS

            @pl.when(j < _NCHUNK)
            def _wb(j=j, gi=gi):
                pltpu.sync_copy(acc.at[pl.ds(j * _CK, _CK)],
                                out_hbm.at[c, pl.ds(gi * _N + j * _CK, _CK)])


_DEPT = 2 * _E // _NW
_DSEG = 800
_DNCH = 2 * _N // _DSEG


@functools.partial(
    pl.kernel,
    out_type=jax.ShapeDtypeStruct((_NC * 2 * _N,), jnp.float32),
    mesh=_mesh,
    compiler_params=pltpu.CompilerParams(needs_layout_passes=False),
    scratch_types=[
        pltpu.VMEM((_DEPT // 400, 400), jnp.int32),
        pltpu.VMEM((2 * _N,), jnp.float32),
        pltpu.VMEM((_NS * _DSEG,), jnp.float32),
        pltpu.VMEM((_DSEG,), jnp.float32),
        pltpu.VMEM_SHARED((_NS * 2 * _N,), jnp.float32),
    ],
)
def _sc_degree(dst_hbm, out_hbm, dstv, hist, red, outv, spbuf):
    c = lax.axis_index("c")
    s = lax.axis_index("s")
    wid = s * _NC + c
    zeros16 = jnp.zeros((16,), jnp.float32)
    ones16 = jnp.ones((16,), jnp.float32)

    def _zrow(r, carry):
        hist[pl.ds(r * 16, 16)] = zeros16
        return carry
    lax.fori_loop(0, 2 * _N // 16, _zrow, 0)

    pltpu.sync_copy(dst_hbm.at[wid], dstv)

    def _count(r, carry):
        for k in range(400 // 16):
            idx = dstv[r, pl.ds(k * 16, 16)]
            plsc.addupdate_scatter(hist, [idx], ones16)
        return carry
    lax.fori_loop(0, _DEPT // 400, _count, 0)

    pltpu.sync_copy(hist, spbuf.at[pl.ds(s * 2 * _N, 2 * _N)])
    plsc.subcore_barrier()

    for rep in range(2):
        j = s + rep * _NS

        @pl.when(j < _DNCH)
        def _reduce(j=j):
            for t in range(_NS):
                pltpu.sync_copy(
                    spbuf.at[pl.ds(t * 2 * _N + j * _DSEG, _DSEG)],
                    red.at[pl.ds(t * _DSEG, _DSEG)])

            def _sum(k, carry):
                v = red[pl.ds(k * 16, 16)]
                for t in range(1, _NS):
                    v = v + red[pl.ds(t * _DSEG + k * 16, 16)]
                outv[pl.ds(k * 16, 16)] = v
                return carry
            lax.fori_loop(0, _DSEG // 16, _sum, 0)
            pltpu.sync_copy(
                outv, out_hbm.at[pl.ds(c * 2 * _N + j * _DSEG, _DSEG)])


def _tc_prep_body(degp_ref, x2_ref, encW_ref, encb_ref, W0_ref,
                  dinv_ref, mm_ref, g_ref):
    deg = degp_ref[:, 0:1] + degp_ref[:, 1:2] + 1.0
    dinv = 1.0 / jnp.sqrt(deg)
    dinv_ref[...] = dinv
    h0 = jnp.dot(x2_ref[...], encW_ref[...],
                 preferred_element_type=jnp.float32) + encb_ref[...]
    mm0 = jnp.dot(h0, W0_ref[...], preferred_element_type=jnp.float32)
    mm_ref[...] = mm0
    g_ref[...] = mm0 * dinv


def _tc_conv_body(s_ref, mm_ref, dinv_ref, b_ref, conv_ref, st1_ref, st2_ref):
    dinv = dinv_ref[...]
    conv = ((s_ref[0] + s_ref[1]) * dinv
            + mm_ref[...] * (dinv * dinv) + b_ref[...])
    conv_ref[...] = conv
    lane = lax.broadcasted_iota(jnp.int32, (1, _H), 1)
    psum = jnp.sum(conv)
    psq = jnp.sum(conv * conv)
    st1_ref[0] = jnp.where(lane == 0, psum,
                           jnp.where(lane == 1, psq, 0.0))
    st2_ref[0] = jnp.sum(conv, axis=0, keepdims=True)


def _ln_scale(st1, lo):
    tot = jnp.sum(st1[lo:lo + _NBG, :, 0:1])
    totsq = jnp.sum(st1[lo:lo + _NBG, :, 1:2])
    m = tot * (1.0 / (_N * _H))
    v = totsq * (1.0 / (_N * _H)) - m * m
    return m, 1.0 / jnp.sqrt(v + 1e-5)


def _tc_ln_mm_body(conv_ref, st1_ref, dinv_ref, lnw_ref, lnb_ref, Wn_ref,
                   mmn_ref, g_ref):
    i = pl.program_id(0)
    st1 = st1_ref[...]
    m_k, r_k = _ln_scale(st1, 0)
    m_d, r_d = _ln_scale(st1, _NBG)
    is_k = i < _NBG
    m = jnp.where(is_k, m_k, m_d)
    r = jnp.where(is_k, r_k, r_d)
    y = (conv_ref[...] - m) * r * lnw_ref[...] + lnb_ref[...]
    h = jnp.maximum(y, 0.0)
    mmn = jnp.dot(h, Wn_ref[...], preferred_element_type=jnp.float32)
    mmn_ref[...] = mmn
    g_ref[...] = mmn * dinv_ref[...]


def _tc_head_body(st1_ref, st2_ref, lnw_ref, lnb_ref,
                  hW1_ref, hb1_ref, hW2_ref, hb2_ref, hW3_ref, hb3_ref,
                  out_ref):
    st1 = st1_ref[...]
    pooled = []
    for lo in (0, _NBG):
        m, r = _ln_scale(st1, lo)
        colsum = jnp.sum(st2_ref[lo:lo + _NBG, 0], axis=0, keepdims=True)
        pooled.append((colsum - _N * m) * r * lnw_ref[...]
                      + _N * lnb_ref[...])
    h1 = jnp.maximum(
        jnp.dot(pooled[0], hW1_ref[0:_H], preferred_element_type=jnp.float32)
        + jnp.dot(pooled[1], hW1_ref[_H:2 * _H],
                  preferred_element_type=jnp.float32)
        + hb1_ref[...], 0.0)
    h2 = jnp.maximum(
        jnp.dot(h1, hW2_ref[...], preferred_element_type=jnp.float32)
        + hb2_ref[...], 0.0)
    out_ref[...] = (jnp.dot(h2, hW3_ref[...], preferred_element_type=jnp.float32)
                    + hb3_ref[...])


def _row_block(i):
    return (i, 0)


_vec_spec = pl.BlockSpec((_R, 1), _row_block)
_mat_spec = pl.BlockSpec((_R, _H), _row_block)
_full = pl.BlockSpec(index_map=lambda i: (0, 0))
_full1 = pl.BlockSpec(index_map=lambda i: (0,))


def _tc_prep(degp, x2, enc_W, enc_b, W0):
    return pl.pallas_call(
        _tc_prep_body,
        grid=(_NB,),
        in_specs=[pl.BlockSpec((_R, _NC), _row_block),
                  pl.BlockSpec((_R, _D), _row_block), _full, _full1, _full],
        out_specs=[_vec_spec, _mat_spec, _mat_spec],
        out_shape=[
            jax.ShapeDtypeStruct((2 * _N, 1), _f32),
            jax.ShapeDtypeStruct((2 * _N, _H), _f32),
            jax.ShapeDtypeStruct((2 * _N, _H), _f32),
        ],
    )(degp, x2, enc_W, enc_b, W0)


def _tc_conv(s_all, mm, dinv, b):
    return pl.pallas_call(
        _tc_conv_body,
        grid=(_NB,),
        in_specs=[pl.BlockSpec((_NC, _R, _H), lambda i: (0, i, 0)),
                  _mat_spec, _vec_spec, _full1],
        out_specs=[_mat_spec,
                   pl.BlockSpec((1, 1, _H), lambda i: (i, 0, 0)),
                   pl.BlockSpec((1, 1, _H), lambda i: (i, 0, 0))],
        out_shape=[
            jax.ShapeDtypeStruct((2 * _N, _H), _f32),
            jax.ShapeDtypeStruct((_NB, 1, _H), _f32),
            jax.ShapeDtypeStruct((_NB, 1, _H), _f32),
        ],
    )(s_all, mm, dinv, b)


def _tc_ln_mm(conv, st1, dinv, lnw, lnb, Wn):
    return pl.pallas_call(
        _tc_ln_mm_body,
        grid=(_NB,),
        in_specs=[_mat_spec, pl.BlockSpec(index_map=lambda i: (0, 0, 0)),
                  _vec_spec, _full1, _full1, _full],
        out_specs=[_mat_spec, _mat_spec],
        out_shape=[
            jax.ShapeDtypeStruct((2 * _N, _H), _f32),
            jax.ShapeDtypeStruct((2 * _N, _H), _f32),
        ],
    )(conv, st1, dinv, lnw, lnb, Wn)


def _tc_head(st1, st2, lnw, lnb, hW1, hb1, hW2, hb2, hW3, hb3):
    return pl.pallas_call(
        _tc_head_body,
        out_shape=jax.ShapeDtypeStruct((1, 1), _f32),
    )(st1, st2, lnw, lnb, hW1, hb1, hW2, hb2, hW3, hb3)


def kernel(kernel_x, design_x, enc_W, enc_b, conv_W0, conv_b0, ln_w0, ln_b0,
           conv_W1, conv_b1, ln_w1, ln_b1, conv_W2, conv_b2, ln_w2, ln_b2,
           head_W1, head_b1, head_W2, head_b2, head_W3, head_b3,
           kernel_edge_index, design_edge_index, pragma_count):
    srck = kernel_edge_index[0].reshape(_NW, _NSTEP, _CH)
    dstk = kernel_edge_index[1].reshape(_NW, _NSTEP, _CH)
    srcd = design_edge_index[0].reshape(_NW, _NSTEP, _CH)
    dstd = design_edge_index[1].reshape(_NW, _NSTEP, _CH)

    src2 = jnp.stack([srck, srcd + _N])
    dst2 = jnp.stack([dstk, dstd])
    src2h = src2.reshape(2, _NW, 2, _NSTEP // 2, _CH)
    dst2h = dst2.reshape(2, _NW, 2, _NSTEP // 2, _CH)
    dstdeg = jnp.concatenate(
        [kernel_edge_index[1],
         design_edge_index[1] + _N]).reshape(_NW, _DEPT // 400, 400)

    degp = jnp.transpose(_sc_degree(dstdeg).reshape(_NC, 2 * _N))

    x2 = jnp.concatenate([kernel_x, design_x], axis=0)
    dinv, mm, g = _tc_prep(degp, x2, enc_W, enc_b, conv_W0)

    Ws = (conv_W1, conv_W2)
    bs = (conv_b0, conv_b1, conv_b2)
    lnws = (ln_w0, ln_w1, ln_w2)
    lnbs = (ln_b0, ln_b1, ln_b2)
    for i in range(3):
        s_all = _sc_scatter(g, src2h, dst2h)
        conv, st1, st2 = _tc_conv(s_all, mm, dinv, bs[i])
        if i < 2:
            mm, g = _tc_ln_mm(conv, st1, dinv, lnws[i], lnbs[i], Ws[i])
        else:
            out = _tc_head(st1, st2, lnws[2], lnbs[2],
                           head_W1, head_b1, head_W2, head_b2,
                           head_W3, head_b3)
    return out
